# baseline (speedup 1.0000x reference)
import math

import jax
import jax.numpy as jnp
from jax.experimental import pallas as pl
from jax.experimental.pallas import tpu as pltpu

N = 8192
D = 5
L = 4
B = 1024
NB = N // B

_LOG2E = math.log2(math.e)


CH = 128
NBC = N // CH
NBUF = 4


def _body(ref3_ref, refT3_ref, u_ref, out_ref,
          abig_ref, bbig_ref, us_ref, exc_ref, *gebufs):
    g_refs = gebufs[:NBUF]
    e_refs = gebufs[NBUF:]
    s = pl.program_id(0)

    @pl.when(s == 0)
    def _init():
        for k in range(NB):
            a = ref3_ref[k] * (2.0 * _LOG2E)
            ah = a.astype(jnp.bfloat16)
            al = (a - ah.astype(jnp.float32)).astype(jnp.bfloat16)
            zpad = jnp.zeros((B, 1), jnp.bfloat16)
            abig_ref[k] = jnp.concatenate([ah, ah, al, zpad], axis=1)
            b = refT3_ref[k]
            bh = b.astype(jnp.bfloat16)
            bl = (b - bh.astype(jnp.float32)).astype(jnp.bfloat16)
            zpad2 = jnp.zeros((1, CH), jnp.bfloat16)
            for h in range(B // CH):
                sl = slice(h * CH, (h + 1) * CH)
                bbig_ref[k * (B // CH) + h] = jnp.concatenate(
                    [bh[:, sl], bl[:, sl], bh[:, sl], zpad2], axis=0)
        rT = jnp.concatenate([refT3_ref[k] for k in range(NB)], axis=1)
        sq = jnp.sum(rT * rT, axis=0, keepdims=True)
        ex = jnp.exp2((-_LOG2E) * sq)
        exc = jnp.transpose(ex, (1, 0))
        exc_ref[...] = exc.reshape(NB, B, 1)
        us = (u_ref[...] * exc).astype(jnp.bfloat16)
        us_ref[...] = us.reshape(NBC, CH, L)

    asel = abig_ref[s]
    o = None
    for c in range(NBC):
        g_ref = g_refs[c % NBUF]
        e_ref = e_refs[c % NBUF]
        g_ref[...] = jnp.dot(asel, bbig_ref[c],
                             preferred_element_type=jnp.float32)
        e_ref[...] = jnp.exp2(g_ref[...]).astype(jnp.bfloat16)
        oc = jnp.dot(e_ref[...], us_ref[c],
                     preferred_element_type=jnp.float32)
        o = oc if o is None else o + oc
    out_ref[...] = o * exc_ref[s]


@jax.jit
def kernel(U, ref):
    n, d = ref.shape
    l = U.shape[1]
    ref3 = ref.reshape(NB, B, d)
    refT3 = jnp.transpose(ref3, (0, 2, 1))

    out = pl.pallas_call(
        _body,
        grid=(NB,),
        in_specs=[
            pl.BlockSpec((NB, B, d), lambda s: (0, 0, 0)),
            pl.BlockSpec((NB, d, B), lambda s: (0, 0, 0)),
            pl.BlockSpec((n, l), lambda s: (0, 0)),
        ],
        out_specs=pl.BlockSpec((B, l), lambda s: (s, 0)),
        out_shape=jax.ShapeDtypeStruct((n, l), jnp.float32),
        scratch_shapes=(
            [pltpu.VMEM((NB, B, 16), jnp.bfloat16),
             pltpu.VMEM((NBC, 16, CH), jnp.bfloat16),
             pltpu.VMEM((NBC, CH, l), jnp.bfloat16),
             pltpu.VMEM((NB, B, 1), jnp.float32)]
            + [pltpu.VMEM((B, CH), jnp.float32) for _ in range(NBUF)]
            + [pltpu.VMEM((B, CH), jnp.bfloat16) for _ in range(NBUF)]
        ),
    )(ref3, refT3, U)
    return out


# speedup vs baseline: 1.2969x; 1.2969x over previous
import math

import jax
import jax.numpy as jnp
import numpy as np
from jax.experimental import pallas as pl
from jax.experimental.pallas import tpu as pltpu

N = 8192
D = 5
L = 4
B = 1024
NB = N // B
CH = 256
NCH = B // CH
NBUF = 4
TRASH = NB

_LOG2E = math.log2(math.e)

_PAIRS = [(i, j) for i in range(NB) for j in range(i, NB)]
NP_ = len(_PAIRS)
GRID = NP_ // 2


def _make_tab():
    tab = np.zeros((3, NP_), dtype=np.int32)
    for q, (i, j) in enumerate(_PAIRS):
        tab[0, q] = i
        tab[1, q] = j
        tab[2, q] = j if j != i else TRASH
    return tab


_TAB = _make_tab()


def _body(tab_ref, ref3_ref, refT3_ref, u_ref, out_ref,
          abig_ref, bbig_ref, us_ref, usT_ref, exc_ref,
          acc1_ref, accT_ref, *gebufs):
    g_refs = gebufs[:NBUF]
    e_refs = gebufs[NBUF:]
    s = pl.program_id(0)

    @pl.when(s == 0)
    def _init():
        for k in range(NB):
            a = ref3_ref[k] * (2.0 * _LOG2E)
            ah = a.astype(jnp.bfloat16)
            al = (a - ah.astype(jnp.float32)).astype(jnp.bfloat16)
            zpad = jnp.zeros((B, 1), jnp.bfloat16)
            abig_ref[k] = jnp.concatenate([ah, ah, al, zpad], axis=1)
            b = refT3_ref[k]
            bh = b.astype(jnp.bfloat16)
            bl = (b - bh.astype(jnp.float32)).astype(jnp.bfloat16)
            zpad2 = jnp.zeros((1, CH), jnp.bfloat16)
            for h in range(NCH):
                sl = slice(h * CH, (h + 1) * CH)
                bbig_ref[k, h] = jnp.concatenate(
                    [bh[:, sl], bl[:, sl], bh[:, sl], zpad2], axis=0)
        rT = jnp.concatenate([refT3_ref[k] for k in range(NB)], axis=1)
        sq = jnp.sum(rT * rT, axis=0, keepdims=True)
        ex = jnp.exp2((-_LOG2E) * sq)
        exc_ref[...] = ex
        excol = jnp.transpose(ex, (1, 0))
        us = (u_ref[...] * excol).astype(jnp.bfloat16)
        us_ref[...] = us.reshape(NB, NCH, CH, L)
        usT = jnp.transpose(us, (1, 0))
        for k in range(NB):
            usT_ref[k] = usT[:, k * B:(k + 1) * B]
        acc1_ref[...] = jnp.zeros((NB + 1, B, L), jnp.float32)
        accT_ref[...] = jnp.zeros((NB + 1, L, B), jnp.float32)

    for t in range(2):
        q = 2 * s + t
        mi = tab_ref[0, q]
        mj = tab_ref[1, q]
        mjc = tab_ref[2, q]
        asel = abig_ref[mi]
        usti = usT_ref[mi]
        o = None
        for h in range(NCH):
            cbuf = (t * NCH + h) % NBUF
            g_ref = g_refs[cbuf]
            e_ref = e_refs[cbuf]
            g_ref[...] = jnp.dot(asel, bbig_ref[mj, h],
                                 preferred_element_type=jnp.float32)
            e_ref[...] = jnp.exp2(g_ref[...]).astype(jnp.bfloat16)
            oc = jnp.dot(e_ref[...], us_ref[mj, h],
                         preferred_element_type=jnp.float32)
            o = oc if o is None else o + oc
            cT = jnp.dot(usti, e_ref[...],
                         preferred_element_type=jnp.float32)
            accT_ref[mjc, :, h * CH:(h + 1) * CH] += cT
        acc1_ref[mi] += o

    @pl.when(s == GRID - 1)
    def _finalize():
        v = acc1_ref[0:NB].reshape(N, L)
        t2 = jnp.transpose(accT_ref[0:NB], (0, 2, 1)).reshape(N, L)
        excol = jnp.transpose(exc_ref[...], (1, 0))
        out_ref[...] = (v + t2) * excol


@jax.jit
def kernel(U, ref):
    n, d = ref.shape
    l = U.shape[1]
    ref3 = ref.reshape(NB, B, d)
    refT3 = jnp.transpose(ref3, (0, 2, 1))

    out = pl.pallas_call(
        _body,
        grid=(GRID,),
        in_specs=[
            pl.BlockSpec(memory_space=pltpu.SMEM),
            pl.BlockSpec((NB, B, d), lambda s: (0, 0, 0)),
            pl.BlockSpec((NB, d, B), lambda s: (0, 0, 0)),
            pl.BlockSpec((n, l), lambda s: (0, 0)),
        ],
        out_specs=pl.BlockSpec((n, l), lambda s: (0, 0)),
        out_shape=jax.ShapeDtypeStruct((n, l), jnp.float32),
        scratch_shapes=(
            [pltpu.VMEM((NB, B, 16), jnp.bfloat16),
             pltpu.VMEM((NB, NCH, 16, CH), jnp.bfloat16),
             pltpu.VMEM((NB, NCH, CH, l), jnp.bfloat16),
             pltpu.VMEM((NB, l, B), jnp.bfloat16),
             pltpu.VMEM((1, n), jnp.float32),
             pltpu.VMEM((NB + 1, B, l), jnp.float32),
             pltpu.VMEM((NB + 1, l, B), jnp.float32)]
            + [pltpu.VMEM((B, CH), jnp.float32) for _ in range(NBUF)]
            + [pltpu.VMEM((B, CH), jnp.bfloat16) for _ in range(NBUF)]
        ),
    )(jnp.asarray(_TAB), ref3, refT3, U)
    return out
